# baseline (speedup 1.0000x reference)
import functools

import jax
import jax.numpy as jnp
from jax import lax
from jax.experimental import pallas as pl
from jax.experimental.pallas import tpu as pltpu
from jax.experimental.pallas import tpu_sc as plsc

_LANES = 16
_CHUNK_ROWS = 32


def _make_sc_kernel(batch, seq_len, d_model, pos_rows):
    info = plsc.get_sparse_core_info()
    nc, ns = info.num_cores, info.num_subcores
    nw = nc * ns
    total_rows = batch * seq_len
    rows_per_w = total_rows // nw
    c = _CHUNK_ROWS
    chunk_elems = c * d_model
    pos_elems = pos_rows * d_model
    nchunks = rows_per_w // c
    mesh = plsc.VectorSubcoreMesh(core_axis_name="c", subcore_axis_name="s")

    @functools.partial(
        pl.kernel,
        mesh=mesh,
        out_type=jax.ShapeDtypeStruct((total_rows * d_model,), jnp.float32),
        scratch_types=[
            pltpu.VMEM((chunk_elems,), jnp.float32),
            pltpu.VMEM((chunk_elems,), jnp.float32),
        ],
    )
    def sc_add(x_hbm, pos_hbm, out_hbm, x_v, pos_v):
        wid = lax.axis_index("s") * nc + lax.axis_index("c")
        base_elem = wid * (rows_per_w * d_model)

        def chunk_body(j, carry):
            e0 = base_elem + j * chunk_elems
            p0 = lax.rem(e0, pos_elems)
            pltpu.sync_copy(x_hbm.at[pl.ds(e0, chunk_elems)], x_v)
            pltpu.sync_copy(pos_hbm.at[pl.ds(p0, chunk_elems)], pos_v)

            def add_body(i, carry2):
                s = pl.ds(i * _LANES, _LANES)
                x_v[s] = x_v[s] + pos_v[s]
                return carry2

            lax.fori_loop(0, chunk_elems // _LANES, add_body, 0)
            pltpu.sync_copy(x_v, out_hbm.at[pl.ds(e0, chunk_elems)])
            return carry

        lax.fori_loop(0, nchunks, chunk_body, 0)

    return sc_add


def kernel(x, pos_weight):
    batch, seq_len, d_model = x.shape
    pos = pos_weight[:seq_len]
    sc = _make_sc_kernel(batch, seq_len, d_model, seq_len)
    out = sc(x.reshape(-1), pos.reshape(-1))
    return out.reshape(x.shape)


# speedup vs baseline: 1.3553x; 1.3553x over previous
import functools

import jax
import jax.numpy as jnp
from jax import lax
from jax.experimental import pallas as pl
from jax.experimental.pallas import tpu as pltpu
from jax.experimental.pallas import tpu_sc as plsc

_LANES = 16
_CHUNK_ROWS = 32
_UNROLL = 8


def _make_sc_kernel(batch, seq_len, d_model, pos_rows):
    info = plsc.get_sparse_core_info()
    nc, ns = info.num_cores, info.num_subcores
    nw = nc * ns
    total_rows = batch * seq_len
    rows_per_w = total_rows // nw
    c = _CHUNK_ROWS
    chunk_elems = c * d_model
    pos_elems = pos_rows * d_model
    nchunks = rows_per_w // c
    mesh = plsc.VectorSubcoreMesh(core_axis_name="c", subcore_axis_name="s")

    @functools.partial(
        pl.kernel,
        mesh=mesh,
        out_type=jax.ShapeDtypeStruct((total_rows * d_model,), jnp.float32),
        scratch_types=[
            pltpu.VMEM((chunk_elems,), jnp.float32),
            pltpu.VMEM((chunk_elems,), jnp.float32),
        ],
    )
    def sc_add(x_hbm, pos_hbm, out_hbm, x_v, pos_v):
        wid = lax.axis_index("s") * nc + lax.axis_index("c")
        base_elem = wid * (rows_per_w * d_model)

        def chunk_body(j, carry):
            e0 = base_elem + j * chunk_elems
            p0 = lax.rem(e0, pos_elems)
            pltpu.sync_copy(x_hbm.at[pl.ds(e0, chunk_elems)], x_v)
            pltpu.sync_copy(pos_hbm.at[pl.ds(p0, chunk_elems)], pos_v)

            def add_body(i, carry2):
                base = i * (_LANES * _UNROLL)
                for u in range(_UNROLL):
                    s = pl.ds(base + u * _LANES, _LANES)
                    x_v[s] = x_v[s] + pos_v[s]
                return carry2

            lax.fori_loop(0, chunk_elems // (_LANES * _UNROLL), add_body, 0)
            pltpu.sync_copy(x_v, out_hbm.at[pl.ds(e0, chunk_elems)])
            return carry

        lax.fori_loop(0, nchunks, chunk_body, 0)

    return sc_add


def kernel(x, pos_weight):
    batch, seq_len, d_model = x.shape
    pos = pos_weight[:seq_len]
    sc = _make_sc_kernel(batch, seq_len, d_model, seq_len)
    out = sc(x.reshape(-1), pos.reshape(-1))
    return out.reshape(x.shape)
